# baseline (speedup 1.0000x reference)
import functools

import jax
import jax.numpy as jnp
from jax import lax
from jax.experimental import pallas as pl
from jax.experimental.pallas import tpu as pltpu
from jax.experimental.pallas import tpu_sc as plsc

N_NODES = 10000
IN_DIM = 128
OUT_DIM = 128
N_EDGES = 320000

NC = 2
NS = 16
NW = NC * NS
LANES = 16
CHUNK = 128
K = 80
E_PAD = NW * K * CHUNK
ROWS_PER_TILE = 624
SLAB = ((0, 128), (128, 128), (256, 128), (384, 128), (512, 112))


def _mm_body(x_ref, w_ref, o_ref):
    o_ref[...] = jnp.dot(x_ref[...], w_ref[...],
                         preferred_element_type=jnp.float32)


def _matmul(x, W):
    m = x.shape[0]
    bm = 1000
    return pl.pallas_call(
        _mm_body,
        grid=(m // bm,),
        in_specs=[
            pl.BlockSpec((bm, IN_DIM), lambda i: (i, 0)),
            pl.BlockSpec((IN_DIM, OUT_DIM), lambda i: (0, 0)),
        ],
        out_specs=pl.BlockSpec((bm, OUT_DIM), lambda i: (i, 0)),
        out_shape=jax.ShapeDtypeStruct((m, OUT_DIM), jnp.float32),
    )(x, W)


def _sc_body(xh_hbm, src_hbm, dst_hbm, ew_hbm, out_hbm,
             sidx0, sidx1, didx0, didx1, ewv0, ewv1, rows0, rows1, acc,
             sg0, sg1, ss0, ss1):
    cid = lax.axis_index("c")
    sid = lax.axis_index("s")
    wid = cid * NS + sid
    sidx = (sidx0, sidx1)
    didx = (didx0, didx1)
    ewv = (ewv0, ewv1)
    rows = (rows0, rows1)
    sg = (sg0, sg1)
    ss = (ss0, ss1)

    zero = jnp.zeros((LANES,), jnp.float32)

    @pl.loop(0, CHUNK)
    def _zrow(r):
        for c in range(OUT_DIM // LANES):
            rows0[r, pl.ds(c * LANES, LANES)] = zero
    base = sid * ROWS_PER_TILE
    for off, n in SLAB:
        pltpu.sync_copy(rows0.at[pl.ds(0, n)], acc.at[pl.ds(base + off, n)])

    @pl.when(sid == NS - 1)
    def _zero_tail():
        pltpu.sync_copy(rows0.at[pl.ds(0, N_NODES - NS * ROWS_PER_TILE)],
                        acc.at[pl.ds(NS * ROWS_PER_TILE,
                                     N_NODES - NS * ROWS_PER_TILE)])
    plsc.subcore_barrier()

    ebase = wid * K * CHUNK

    def _stage(c, b):
        e0 = ebase + c * CHUNK
        pltpu.sync_copy(src_hbm.at[pl.ds(e0, CHUNK)], sidx[b])
        pltpu.sync_copy(dst_hbm.at[pl.ds(e0, CHUNK)], didx[b])
        pltpu.sync_copy(ew_hbm.at[pl.ds(e0 * LANES, CHUNK * LANES)], ewv[b])
        pltpu.make_async_copy(xh_hbm.at[sidx[b]], rows[b], sg[b]).start()

    _stage(0, 0)

    @pl.loop(0, K // 2)
    def _pair(j):
        for b in (0, 1):
            nb = 1 - b
            c = 2 * j + b

            @pl.when(c + 1 < K)
            def _prefetch():
                @pl.when(c >= 1)
                def _drain_prev_scatter():
                    pltpu.make_async_copy(rows[nb], acc.at[didx[nb]],
                                          ss[nb]).wait()
                _stage(c + 1, nb)

            pltpu.make_async_copy(xh_hbm.at[sidx[b]], rows[b], sg[b]).wait()

            @pl.loop(0, CHUNK, unroll=2)
            def _row(r):
                w = ewv[b][pl.ds(r * LANES, LANES)]
                for cc in range(OUT_DIM // LANES):
                    sl = pl.ds(cc * LANES, LANES)
                    rows[b][r, sl] = rows[b][r, sl] * w

            pltpu.make_async_copy(rows[b], acc.at[didx[b]],
                                  ss[b]).start(add=True)

    pltpu.make_async_copy(rows0, acc.at[didx0], ss0).wait()
    pltpu.make_async_copy(rows1, acc.at[didx1], ss1).wait()

    plsc.subcore_barrier()
    for off, n in SLAB:
        pltpu.sync_copy(acc.at[pl.ds(base + off, n)],
                        out_hbm.at[cid, pl.ds(base + off, n)])

    @pl.when(sid == NS - 1)
    def _write_tail():
        tail0 = NS * ROWS_PER_TILE
        ntail = N_NODES - tail0
        pltpu.sync_copy(acc.at[pl.ds(tail0, ntail)],
                        out_hbm.at[cid, pl.ds(tail0, ntail)])


_sc_call = pl.kernel(
    _sc_body,
    out_type=jax.ShapeDtypeStruct((NC, N_NODES, OUT_DIM), jnp.float32),
    mesh=plsc.VectorSubcoreMesh(core_axis_name="c", subcore_axis_name="s"),
    scratch_types=[
        pltpu.VMEM((CHUNK,), jnp.int32),
        pltpu.VMEM((CHUNK,), jnp.int32),
        pltpu.VMEM((CHUNK,), jnp.int32),
        pltpu.VMEM((CHUNK,), jnp.int32),
        pltpu.VMEM((CHUNK * LANES,), jnp.float32),
        pltpu.VMEM((CHUNK * LANES,), jnp.float32),
        pltpu.VMEM((CHUNK, OUT_DIM), jnp.float32),
        pltpu.VMEM((CHUNK, OUT_DIM), jnp.float32),
        pltpu.VMEM_SHARED((N_NODES, OUT_DIM), jnp.float32),
        pltpu.SemaphoreType.DMA,
        pltpu.SemaphoreType.DMA,
        pltpu.SemaphoreType.DMA,
        pltpu.SemaphoreType.DMA,
    ],
)


def _sc_edges(xh, src1, dst1, ew1):
    return _sc_call(xh, src1, dst1, ew1)


def _fin_body(a_ref, p_ref, o_ref):
    s = p_ref[0] + p_ref[1]
    slope = a_ref[0, 0]
    o_ref[...] = jnp.where(s > 0, s, slope * s)


def _finish(a2, partial):
    bm = 1000
    return pl.pallas_call(
        _fin_body,
        grid=(N_NODES // bm,),
        in_specs=[
            pl.BlockSpec(memory_space=pltpu.SMEM),
            pl.BlockSpec((NC, bm, OUT_DIM), lambda i: (0, i, 0)),
        ],
        out_specs=pl.BlockSpec((bm, OUT_DIM), lambda i: (i, 0)),
        out_shape=jax.ShapeDtypeStruct((N_NODES, OUT_DIM), jnp.float32),
    )(a2, partial)


@jax.jit
def kernel(x, edge_index, edge_weight, W, a):
    xh = _matmul(x, W)

    dst = edge_index[0].astype(jnp.int32)
    src = edge_index[1].astype(jnp.int32)
    ew = edge_weight.astype(jnp.float32)
    pad = E_PAD - N_EDGES
    src1 = jnp.pad(src, (0, pad))
    dst1 = jnp.pad(dst, (0, pad))
    ew1 = jnp.pad(ew, (0, pad))
    ew16 = jnp.broadcast_to(ew1[:, None], (E_PAD, LANES)).reshape(-1)

    partial = _sc_edges(xh, src1, dst1, ew16)

    a2 = jnp.reshape(a, (1, 1)).astype(jnp.float32)
    return _finish(a2, partial)
